# Initial kernel scaffold; baseline (speedup 1.0000x reference)
import jax
import jax.numpy as jnp
from jax.experimental import pallas as pl


def kernel(z, codebook):
    raise NotImplementedError("write your pallas kernel here")


# speedup vs baseline: 1.2696x; 1.2696x over previous
import functools

import jax
import jax.numpy as jnp
from jax import lax
from jax.experimental import pallas as pl
from jax.experimental.pallas import tpu as pltpu
from jax.experimental.pallas import tpu_sc as plsc

_VOCAB = 8192
_EMBED = 256
_ROWS = 8192
_TM = 512
_TV = 1024
_RT = _ROWS // _TM
_VT = _VOCAB // _TV


def _argmin_body(z_ref, zn_ref, cn_ref, cb_ref, idx_ref, loss_ref, bv_ref, bi_ref):
    v = pl.program_id(1)

    @pl.when(v == 0)
    def _init():
        bv_ref[...] = jnp.full((_TM, 1), jnp.inf, jnp.float32)
        bi_ref[...] = jnp.zeros((_TM, 1), jnp.int32)

    z = z_ref[...]
    cb = cb_ref[...]
    mm = lax.dot_general(z, cb, (((1,), (1,)), ((), ())),
                         preferred_element_type=jnp.float32)
    cn_blk = cn_ref[:, pl.ds(v * _TV, _TV)]
    d = (zn_ref[...] + cn_blk) - 2.0 * mm
    m = jnp.min(d, axis=1, keepdims=True)
    col = lax.broadcasted_iota(jnp.int32, (_TM, _TV), 1)
    li = jnp.min(jnp.where(d == m, col, _TV), axis=1, keepdims=True) + v * _TV
    better = m < bv_ref[...]
    bi_ref[...] = jnp.where(better, li, bi_ref[...])
    bv_ref[...] = jnp.where(better, m, bv_ref[...])
    idx_ref[...] = bi_ref[...]
    loss_ref[...] = jnp.sum(bv_ref[...]).reshape(1, 1, 1)


def _run_argmin(z2, zn, cn2, cb):
    return pl.pallas_call(
        _argmin_body,
        grid=(_RT, _VT),
        in_specs=[
            pl.BlockSpec((_TM, _EMBED), lambda r, v: (r, 0)),
            pl.BlockSpec((_TM, 1), lambda r, v: (r, 0)),
            pl.BlockSpec((1, _VOCAB), lambda r, v: (0, 0)),
            pl.BlockSpec((_TV, _EMBED), lambda r, v: (v, 0)),
        ],
        out_specs=[
            pl.BlockSpec((_TM, 1), lambda r, v: (r, 0)),
            pl.BlockSpec((1, 1, 1), lambda r, v: (r, 0, 0)),
        ],
        out_shape=[
            jax.ShapeDtypeStruct((_ROWS, 1), jnp.int32),
            jax.ShapeDtypeStruct((_RT, 1, 1), jnp.float32),
        ],
        scratch_shapes=[
            pltpu.VMEM((_TM, 1), jnp.float32),
            pltpu.VMEM((_TM, 1), jnp.int32),
        ],
    )(z2, zn, cn2, cb)


def _ppl_body(idx_ref, out_ref):
    idx = idx_ref[...]
    c = jnp.zeros(idx.shape, jnp.int32)
    for b in range(8):
        c = c + (idx == idx[b:b + 1, :]).astype(jnp.int32)
    p = c.astype(jnp.float32) * 0.125
    ent = jnp.sum(jnp.log(p + 1e-10)) * 0.125
    out_ref[...] = jnp.full((1, 1), jnp.exp(-ent), jnp.float32)


def _run_ppl(idx8):
    return pl.pallas_call(
        _ppl_body,
        out_shape=jax.ShapeDtypeStruct((1, 1), jnp.float32),
    )(idx8)


_NC = 2
_NS = 16
_NW = _NC * _NS
_BPW = _ROWS // _NW
_CH = 128
_NCH = _BPW // _CH

@functools.cache
def _make_sc_gather():
    mesh = plsc.VectorSubcoreMesh(core_axis_name="c", subcore_axis_name="s")

    @functools.partial(
        pl.kernel,
        mesh=mesh,
        out_type=jax.ShapeDtypeStruct((_ROWS, _EMBED), jnp.float32),
        scratch_types=[
            pltpu.VMEM((_CH,), jnp.int32),
            pltpu.VMEM((_CH,), jnp.int32),
            pltpu.VMEM((_CH, _EMBED), jnp.float32),
            pltpu.VMEM((_CH, _EMBED), jnp.float32),
            pltpu.SemaphoreType.DMA,
            pltpu.SemaphoreType.DMA,
        ],
    )
    def _sc_gather(table_hbm, idx_hbm, out_hbm, idx0, idx1, rows0, rows1, sem0, sem1):
        wid = lax.axis_index("s") * _NC + lax.axis_index("c")
        base = wid * _BPW
        idx_bufs = (idx0, idx1)
        row_bufs = (rows0, rows1)
        sems = (sem0, sem1)
        copies = []
        for k in range(_NCH):
            pltpu.sync_copy(idx_hbm.at[pl.ds(base + k * _CH, _CH)], idx_bufs[k])
            copies.append(pltpu.async_copy(table_hbm.at[idx_bufs[k]], row_bufs[k], sems[k]))
        for k in range(_NCH):
            copies[k].wait()
            pltpu.sync_copy(row_bufs[k], out_hbm.at[pl.ds(base + k * _CH, _CH)])

    return _sc_gather


def kernel(z, codebook):
    B, C, H, W = z.shape
    z_flat = jnp.transpose(z, (0, 2, 3, 1)).reshape(B, H * W, C).astype(jnp.float32)
    cb = codebook.astype(jnp.float32)
    zn = jnp.sum(z_flat ** 2, axis=-1, keepdims=True)
    cn = jnp.sum(cb ** 2, axis=-1)

    z2 = z_flat.reshape(B * H * W, C)
    idx2, loss_parts = _run_argmin(z2, zn.reshape(B * H * W, 1), cn.reshape(1, _VOCAB), cb)

    idx_flat = idx2.reshape(B * H * W)
    zq_flat = _make_sc_gather()(cb, idx_flat)

    perplexity = _run_ppl(idx2.reshape(B, H * W))[0, 0]
    z_q = jnp.transpose(zq_flat.reshape(B, H, W, C), (0, 3, 1, 2))
    commitment_loss = jnp.sum(loss_parts) / (B * C * H * W)
    indices = idx_flat.reshape(B, H, W)
    return z_q, indices, commitment_loss, perplexity


# speedup vs baseline: 1.3405x; 1.0558x over previous
import functools

import jax
import jax.numpy as jnp
from jax import lax
from jax.experimental import pallas as pl
from jax.experimental.pallas import tpu as pltpu
from jax.experimental.pallas import tpu_sc as plsc

_VOCAB = 8192
_EMBED = 256
_ROWS = 8192
_TM = 512
_TV = 1024
_RT = _ROWS // _TM
_VT = _VOCAB // _TV


def _argmin_body(z_ref, zn_ref, cn_ref, colf_ref, cb_ref, idx_ref, loss_ref,
                 bv_ref, bi_ref):
    v = pl.program_id(1)

    zt = z_ref[0]
    zn = zn_ref[...]

    val = None
    _W = 256
    for k in range(_TV // _W):
        off = v * _TV + k * _W
        cbk = cb_ref[k * _W:(k + 1) * _W, :]
        mmk = lax.dot_general(zt, cbk, (((0,), (1,)), ((), ())),
                              preferred_element_type=jnp.float32)
        cnk = cn_ref[:, pl.ds(off, _W)]
        colk = colf_ref[:, pl.ds(off, _W)]
        dc = (zn + cnk) + mmk
        dk0, dk1 = dc[:, :128], dc[:, 128:]
        ck0, ck1 = colk[:, :128], colk[:, 128:]
        cf = dk1 < dk0
        dk = jnp.where(cf, dk1, dk0)
        ik = jnp.where(cf, jnp.broadcast_to(ck1, (_TM, 128)),
                       jnp.broadcast_to(ck0, (_TM, 128)))
        if val is None:
            val = dk
            idx = ik
        else:
            c = dk < val
            idx = jnp.where(c, ik, idx)
            val = jnp.where(c, dk, val)

    @pl.when(v == 0)
    def _first():
        bv_ref[...] = val
        bi_ref[...] = idx

    @pl.when(v > 0)
    def _merge():
        c = val < bv_ref[...]
        bi_ref[...] = jnp.where(c, idx, bi_ref[...])
        bv_ref[...] = jnp.where(c, val, bv_ref[...])

    @pl.when(v == _VT - 1)
    def _finish():
        bv = bv_ref[...]
        bi = bi_ref[...]
        m = jnp.min(bv, axis=1, keepdims=True)
        lif = jnp.min(jnp.where(bv == m, bi, jnp.inf), axis=1, keepdims=True)
        idx_ref[...] = lif.astype(jnp.int32)
        loss_ref[...] = jnp.sum(m).reshape(1, 1, 1)


def _run_argmin(z2, zn, cn2, cb):
    colf = jnp.arange(_VOCAB, dtype=jnp.float32).reshape(1, _VOCAB)
    return pl.pallas_call(
        _argmin_body,
        grid=(_RT, _VT),
        in_specs=[
            pl.BlockSpec((1, _EMBED, _TM), lambda r, v: (r // 2, 0, r % 2)),
            pl.BlockSpec((_TM, 1), lambda r, v: (r, 0)),
            pl.BlockSpec((1, _VOCAB), lambda r, v: (0, 0)),
            pl.BlockSpec((1, _VOCAB), lambda r, v: (0, 0)),
            pl.BlockSpec((_TV, _EMBED), lambda r, v: (v, 0)),
        ],
        out_specs=[
            pl.BlockSpec((_TM, 1), lambda r, v: (r, 0)),
            pl.BlockSpec((1, 1, 1), lambda r, v: (r, 0, 0)),
        ],
        out_shape=[
            jax.ShapeDtypeStruct((_ROWS, 1), jnp.int32),
            jax.ShapeDtypeStruct((_RT, 1, 1), jnp.float32),
        ],
        scratch_shapes=[
            pltpu.VMEM((_TM, 128), jnp.float32),
            pltpu.VMEM((_TM, 128), jnp.float32),
        ],
    )(z2, zn, cn2, colf, cb)


def _ppl_body(idx_ref, out_ref):
    idx = idx_ref[...]
    c = jnp.zeros(idx.shape, jnp.int32)
    for b in range(8):
        c = c + (idx == idx[b:b + 1, :]).astype(jnp.int32)
    p = c.astype(jnp.float32) * 0.125
    ent = jnp.sum(jnp.log(p + 1e-10)) * 0.125
    out_ref[...] = jnp.full((1, 1), jnp.exp(-ent), jnp.float32)


def _run_ppl(idx8):
    return pl.pallas_call(
        _ppl_body,
        out_shape=jax.ShapeDtypeStruct((1, 1), jnp.float32),
    )(idx8)


_NC = 2
_NS = 16
_NW = _NC * _NS
_BPW = _ROWS // _NW
_CH = 128
_NCH = _BPW // _CH

@functools.cache
def _make_sc_gather():
    mesh = plsc.VectorSubcoreMesh(core_axis_name="c", subcore_axis_name="s")

    @functools.partial(
        pl.kernel,
        mesh=mesh,
        out_type=jax.ShapeDtypeStruct((_ROWS, _EMBED), jnp.float32),
        scratch_types=[
            pltpu.VMEM((_CH,), jnp.int32),
            pltpu.VMEM((_CH,), jnp.int32),
            pltpu.VMEM((_CH, _EMBED), jnp.float32),
            pltpu.VMEM((_CH, _EMBED), jnp.float32),
            pltpu.SemaphoreType.DMA,
            pltpu.SemaphoreType.DMA,
        ],
    )
    def _sc_gather(table_hbm, idx_hbm, out_hbm, idx0, idx1, rows0, rows1, sem0, sem1):
        wid = lax.axis_index("s") * _NC + lax.axis_index("c")
        base = wid * _BPW
        idx_bufs = (idx0, idx1)
        row_bufs = (rows0, rows1)
        sems = (sem0, sem1)
        copies = []
        for k in range(_NCH):
            pltpu.sync_copy(idx_hbm.at[pl.ds(base + k * _CH, _CH)], idx_bufs[k])
            copies.append(pltpu.async_copy(table_hbm.at[idx_bufs[k]], row_bufs[k], sems[k]))
        for k in range(_NCH):
            copies[k].wait()
            pltpu.sync_copy(row_bufs[k], out_hbm.at[pl.ds(base + k * _CH, _CH)])

    return _sc_gather


def kernel(z, codebook):
    B, C, H, W = z.shape
    z_flat = jnp.transpose(z, (0, 2, 3, 1)).reshape(B, H * W, C).astype(jnp.float32)
    cb = codebook.astype(jnp.float32)
    zn = jnp.sum(z_flat ** 2, axis=-1, keepdims=True)
    cn = jnp.sum(cb ** 2, axis=-1)

    m2z = (-2.0 * z.astype(jnp.float32)).reshape(B, C, H * W)
    idx2, loss_parts = _run_argmin(m2z, zn.reshape(B * H * W, 1), cn.reshape(1, _VOCAB), cb)

    idx_flat = idx2.reshape(B * H * W)
    zq_flat = _make_sc_gather()(cb, idx_flat)

    perplexity = _run_ppl(idx2.reshape(B, H * W))[0, 0]
    z_q = jnp.transpose(zq_flat.reshape(B, H, W, C), (0, 3, 1, 2))
    commitment_loss = jnp.sum(loss_parts) / (B * C * H * W)
    indices = idx_flat.reshape(B, H, W)
    return z_q, indices, commitment_loss, perplexity


# speedup vs baseline: 1.6843x; 1.2565x over previous
import functools

import jax
import jax.numpy as jnp
from jax import lax
from jax.experimental import pallas as pl
from jax.experimental.pallas import tpu as pltpu
from jax.experimental.pallas import tpu_sc as plsc

_VOCAB = 8192
_EMBED = 256
_ROWS = 8192
_TM = 512
_TV = 2048
_RT = _ROWS // _TM
_VT = _VOCAB // _TV


def _argmin_body(z_ref, zn_ref, cn_ref, colf_ref, cb_ref, idx_ref, loss_ref,
                 bv_ref, bi_ref):
    v = pl.program_id(1)

    zt = z_ref[0]
    zn = zn_ref[...]

    val = None
    _W = 256
    for k in range(_TV // _W):
        off = v * _TV + k * _W
        cbk = cb_ref[k * _W:(k + 1) * _W, :]
        mmk = lax.dot_general(zt, cbk, (((0,), (1,)), ((), ())),
                              preferred_element_type=jnp.float32)
        cnk = cn_ref[:, pl.ds(off, _W)]
        colk = colf_ref[:, pl.ds(off, _W)]
        dc = (zn + cnk) + mmk
        dk0, dk1 = dc[:, :128], dc[:, 128:]
        ck0, ck1 = colk[:, :128], colk[:, 128:]
        cf = dk1 < dk0
        dk = jnp.where(cf, dk1, dk0)
        ik = jnp.where(cf, jnp.broadcast_to(ck1, (_TM, 128)),
                       jnp.broadcast_to(ck0, (_TM, 128)))
        if val is None:
            val = dk
            idx = ik
        else:
            c = dk < val
            idx = jnp.where(c, ik, idx)
            val = jnp.where(c, dk, val)

    @pl.when(v == 0)
    def _first():
        bv_ref[...] = val
        bi_ref[...] = idx

    @pl.when(v > 0)
    def _merge():
        c = val < bv_ref[...]
        bi_ref[...] = jnp.where(c, idx, bi_ref[...])
        bv_ref[...] = jnp.where(c, val, bv_ref[...])

    @pl.when(v == _VT - 1)
    def _finish():
        bv = bv_ref[...]
        bi = bi_ref[...]
        m = jnp.min(bv, axis=1, keepdims=True)
        lif = jnp.min(jnp.where(bv == m, bi, jnp.inf), axis=1, keepdims=True)
        idx_ref[...] = lif.astype(jnp.int32)
        loss_ref[...] = jnp.sum(m).reshape(1, 1, 1)


def _run_argmin(z2, zn, cn2, cb):
    colf = jnp.arange(_VOCAB, dtype=jnp.float32).reshape(1, _VOCAB)
    return pl.pallas_call(
        _argmin_body,
        grid=(_RT, _VT),
        in_specs=[
            pl.BlockSpec((1, _EMBED, _TM), lambda r, v: (r // 2, 0, r % 2)),
            pl.BlockSpec((_TM, 1), lambda r, v: (r, 0)),
            pl.BlockSpec((1, _VOCAB), lambda r, v: (0, 0)),
            pl.BlockSpec((1, _VOCAB), lambda r, v: (0, 0)),
            pl.BlockSpec((_TV, _EMBED), lambda r, v: (v, 0)),
        ],
        out_specs=[
            pl.BlockSpec((_TM, 1), lambda r, v: (r, 0)),
            pl.BlockSpec((1, 1, 1), lambda r, v: (r, 0, 0)),
        ],
        out_shape=[
            jax.ShapeDtypeStruct((_ROWS, 1), jnp.int32),
            jax.ShapeDtypeStruct((_RT, 1, 1), jnp.float32),
        ],
        scratch_shapes=[
            pltpu.VMEM((_TM, 128), jnp.float32),
            pltpu.VMEM((_TM, 128), jnp.float32),
        ],
    )(z2, zn, cn2, colf, cb)


def _ppl_body(idx_ref, out_ref):
    idx = idx_ref[...]
    c = jnp.zeros(idx.shape, jnp.int32)
    for b in range(8):
        c = c + (idx == idx[b:b + 1, :]).astype(jnp.int32)
    p = c.astype(jnp.float32) * 0.125
    ent = jnp.sum(jnp.log(p + 1e-10)) * 0.125
    out_ref[...] = jnp.full((1, 1), jnp.exp(-ent), jnp.float32)


def _run_ppl(idx8):
    return pl.pallas_call(
        _ppl_body,
        out_shape=jax.ShapeDtypeStruct((1, 1), jnp.float32),
    )(idx8)


_NC = 2
_NS = 16
_NW = _NC * _NS
_BPW = _ROWS // _NW
_CH = 128
_NCH = _BPW // _CH

@functools.cache
def _make_sc_gather():
    mesh = plsc.VectorSubcoreMesh(core_axis_name="c", subcore_axis_name="s")

    @functools.partial(
        pl.kernel,
        mesh=mesh,
        out_type=jax.ShapeDtypeStruct((_ROWS, _EMBED), jnp.float32),
        scratch_types=[
            pltpu.VMEM((_CH,), jnp.int32),
            pltpu.VMEM((_CH,), jnp.int32),
            pltpu.VMEM((_CH, _EMBED), jnp.float32),
            pltpu.VMEM((_CH, _EMBED), jnp.float32),
            pltpu.SemaphoreType.DMA,
            pltpu.SemaphoreType.DMA,
        ],
    )
    def _sc_gather(table_hbm, idx_hbm, out_hbm, idx0, idx1, rows0, rows1, sem0, sem1):
        wid = lax.axis_index("s") * _NC + lax.axis_index("c")
        base = wid * _BPW
        idx_bufs = (idx0, idx1)
        row_bufs = (rows0, rows1)
        sems = (sem0, sem1)
        copies = []
        for k in range(_NCH):
            pltpu.sync_copy(idx_hbm.at[pl.ds(base + k * _CH, _CH)], idx_bufs[k])
            copies.append(pltpu.async_copy(table_hbm.at[idx_bufs[k]], row_bufs[k], sems[k]))
        for k in range(_NCH):
            copies[k].wait()
            pltpu.sync_copy(row_bufs[k], out_hbm.at[pl.ds(base + k * _CH, _CH)])

    return _sc_gather


def kernel(z, codebook):
    B, C, H, W = z.shape
    z_flat = jnp.transpose(z, (0, 2, 3, 1)).reshape(B, H * W, C).astype(jnp.float32)
    cb = codebook.astype(jnp.float32)
    zn = jnp.sum(z_flat ** 2, axis=-1, keepdims=True)
    cn = jnp.sum(cb ** 2, axis=-1)

    m2z = (-2.0 * z.astype(jnp.float32)).reshape(B, C, H * W)
    idx2, loss_parts = _run_argmin(m2z, zn.reshape(B * H * W, 1), cn.reshape(1, _VOCAB), cb)

    idx_flat = idx2.reshape(B * H * W)
    zq_flat = _make_sc_gather()(cb, idx_flat)

    perplexity = _run_ppl(idx2.reshape(B, H * W))[0, 0]
    z_q = jnp.transpose(zq_flat.reshape(B, H, W, C), (0, 3, 1, 2))
    commitment_loss = jnp.sum(loss_parts) / (B * C * H * W)
    indices = idx_flat.reshape(B, H, W)
    return z_q, indices, commitment_loss, perplexity


# speedup vs baseline: 2.0667x; 1.2270x over previous
import functools

import jax
import jax.numpy as jnp
from jax import lax
from jax.experimental import pallas as pl
from jax.experimental.pallas import tpu as pltpu
from jax.experimental.pallas import tpu_sc as plsc

_VOCAB = 8192
_EMBED = 256
_ROWS = 8192
_TM = 512
_TV = 8192
_RT = _ROWS // _TM
_VT = _VOCAB // _TV


def _argmin_body(z_ref, zn_ref, cn_ref, colf_ref, cb_ref, idx_ref, loss_ref,
                 bv_ref, bi_ref):
    v = pl.program_id(1)

    zt = z_ref[0]
    zn = zn_ref[...]

    val = None
    _W = 256
    for k in range(_TV // _W):
        off = v * _TV + k * _W
        cbk = cb_ref[k * _W:(k + 1) * _W, :]
        mmk = lax.dot_general(zt, cbk, (((0,), (1,)), ((), ())),
                              preferred_element_type=jnp.float32)
        cnk = cn_ref[:, pl.ds(off, _W)]
        colk = colf_ref[:, pl.ds(off, _W)]
        dc = (zn + cnk) + mmk
        dk0, dk1 = dc[:, :128], dc[:, 128:]
        ck0, ck1 = colk[:, :128], colk[:, 128:]
        cf = dk1 < dk0
        dk = jnp.where(cf, dk1, dk0)
        ik = jnp.where(cf, jnp.broadcast_to(ck1, (_TM, 128)),
                       jnp.broadcast_to(ck0, (_TM, 128)))
        if val is None:
            val = dk
            idx = ik
        else:
            c = dk < val
            idx = jnp.where(c, ik, idx)
            val = jnp.where(c, dk, val)

    @pl.when(v == 0)
    def _first():
        bv_ref[...] = val
        bi_ref[...] = idx

    @pl.when(v > 0)
    def _merge():
        c = val < bv_ref[...]
        bi_ref[...] = jnp.where(c, idx, bi_ref[...])
        bv_ref[...] = jnp.where(c, val, bv_ref[...])

    @pl.when(v == _VT - 1)
    def _finish():
        bv = bv_ref[...]
        bi = bi_ref[...]
        m = jnp.min(bv, axis=1, keepdims=True)
        lif = jnp.min(jnp.where(bv == m, bi, jnp.inf), axis=1, keepdims=True)
        idx_ref[...] = lif.astype(jnp.int32)
        loss_ref[...] = jnp.sum(m).reshape(1, 1, 1)


def _run_argmin(z2, zn, cn2, cb):
    colf = jnp.arange(_VOCAB, dtype=jnp.float32).reshape(1, _VOCAB)
    return pl.pallas_call(
        _argmin_body,
        grid=(_RT, _VT),
        in_specs=[
            pl.BlockSpec((1, _EMBED, _TM), lambda r, v: (r // 2, 0, r % 2)),
            pl.BlockSpec((_TM, 1), lambda r, v: (r, 0)),
            pl.BlockSpec((1, _VOCAB), lambda r, v: (0, 0)),
            pl.BlockSpec((1, _VOCAB), lambda r, v: (0, 0)),
            pl.BlockSpec((_TV, _EMBED), lambda r, v: (v, 0)),
        ],
        out_specs=[
            pl.BlockSpec((_TM, 1), lambda r, v: (r, 0)),
            pl.BlockSpec((1, 1, 1), lambda r, v: (r, 0, 0)),
        ],
        out_shape=[
            jax.ShapeDtypeStruct((_ROWS, 1), jnp.int32),
            jax.ShapeDtypeStruct((_RT, 1, 1), jnp.float32),
        ],
        scratch_shapes=[
            pltpu.VMEM((_TM, 128), jnp.float32),
            pltpu.VMEM((_TM, 128), jnp.float32),
        ],
    )(z2, zn, cn2, colf, cb)


def _ppl_body(idx_ref, out_ref):
    idx = idx_ref[...]
    c = jnp.zeros(idx.shape, jnp.int32)
    for b in range(8):
        c = c + (idx == idx[b:b + 1, :]).astype(jnp.int32)
    p = c.astype(jnp.float32) * 0.125
    ent = jnp.sum(jnp.log(p + 1e-10)) * 0.125
    out_ref[...] = jnp.full((1, 1), jnp.exp(-ent), jnp.float32)


def _run_ppl(idx8):
    return pl.pallas_call(
        _ppl_body,
        out_shape=jax.ShapeDtypeStruct((1, 1), jnp.float32),
    )(idx8)


_NC = 2
_NS = 16
_NW = _NC * _NS
_BPW = _ROWS // _NW
_CH = 128
_NCH = _BPW // _CH

@functools.cache
def _make_sc_gather():
    mesh = plsc.VectorSubcoreMesh(core_axis_name="c", subcore_axis_name="s")

    @functools.partial(
        pl.kernel,
        mesh=mesh,
        out_type=jax.ShapeDtypeStruct((_ROWS, _EMBED), jnp.float32),
        scratch_types=[
            pltpu.VMEM((_CH,), jnp.int32),
            pltpu.VMEM((_CH,), jnp.int32),
            pltpu.VMEM((_CH, _EMBED), jnp.float32),
            pltpu.VMEM((_CH, _EMBED), jnp.float32),
            pltpu.SemaphoreType.DMA,
            pltpu.SemaphoreType.DMA,
        ],
    )
    def _sc_gather(table_hbm, idx_hbm, out_hbm, idx0, idx1, rows0, rows1, sem0, sem1):
        wid = lax.axis_index("s") * _NC + lax.axis_index("c")
        base = wid * _BPW
        idx_bufs = (idx0, idx1)
        row_bufs = (rows0, rows1)
        sems = (sem0, sem1)
        copies = []
        for k in range(_NCH):
            pltpu.sync_copy(idx_hbm.at[pl.ds(base + k * _CH, _CH)], idx_bufs[k])
            copies.append(pltpu.async_copy(table_hbm.at[idx_bufs[k]], row_bufs[k], sems[k]))
        for k in range(_NCH):
            copies[k].wait()
            pltpu.sync_copy(row_bufs[k], out_hbm.at[pl.ds(base + k * _CH, _CH)])

    return _sc_gather


def kernel(z, codebook):
    B, C, H, W = z.shape
    z_flat = jnp.transpose(z, (0, 2, 3, 1)).reshape(B, H * W, C).astype(jnp.float32)
    cb = codebook.astype(jnp.float32)
    zn = jnp.sum(z_flat ** 2, axis=-1, keepdims=True)
    cn = jnp.sum(cb ** 2, axis=-1)

    m2z = (-2.0 * z.astype(jnp.float32)).reshape(B, C, H * W)
    idx2, loss_parts = _run_argmin(m2z, zn.reshape(B * H * W, 1), cn.reshape(1, _VOCAB), cb)

    idx_flat = idx2.reshape(B * H * W)
    zq_flat = _make_sc_gather()(cb, idx_flat)

    perplexity = _run_ppl(idx2.reshape(B, H * W))[0, 0]
    z_q = jnp.transpose(zq_flat.reshape(B, H, W, C), (0, 3, 1, 2))
    commitment_loss = jnp.sum(loss_parts) / (B * C * H * W)
    indices = idx_flat.reshape(B, H, W)
    return z_q, indices, commitment_loss, perplexity


# speedup vs baseline: 2.1983x; 1.0637x over previous
import functools

import jax
import jax.numpy as jnp
from jax import lax
from jax.experimental import pallas as pl
from jax.experimental.pallas import tpu as pltpu
from jax.experimental.pallas import tpu_sc as plsc

_VOCAB = 8192
_EMBED = 256
_ROWS = 8192
_TM = 512
_TV = 8192
_RT = _ROWS // _TM
_VT = _VOCAB // _TV


def _argmin_body(z_ref, zn_ref, cn_ref, colf_ref, cb_ref, idx_ref, loss_ref,
                 bv_ref, bi_ref):
    v = pl.program_id(1)

    zt = z_ref[0] * -2.0
    zn = zn_ref[...]

    val = None
    _W = 256
    for k in range(_TV // _W):
        off = v * _TV + k * _W
        cbk = cb_ref[k * _W:(k + 1) * _W, :]
        mmk = lax.dot_general(zt, cbk, (((0,), (1,)), ((), ())),
                              preferred_element_type=jnp.float32)
        cnk = cn_ref[:, pl.ds(off, _W)]
        colk = colf_ref[:, pl.ds(off, _W)]
        dc = (zn + cnk) + mmk
        dk0, dk1 = dc[:, :128], dc[:, 128:]
        ck0, ck1 = colk[:, :128], colk[:, 128:]
        cf = dk1 < dk0
        dk = jnp.where(cf, dk1, dk0)
        ik = jnp.where(cf, jnp.broadcast_to(ck1, (_TM, 128)),
                       jnp.broadcast_to(ck0, (_TM, 128)))
        if val is None:
            val = dk
            idx = ik
        else:
            c = dk < val
            idx = jnp.where(c, ik, idx)
            val = jnp.where(c, dk, val)

    @pl.when(v == 0)
    def _first():
        bv_ref[...] = val
        bi_ref[...] = idx

    @pl.when(v > 0)
    def _merge():
        c = val < bv_ref[...]
        bi_ref[...] = jnp.where(c, idx, bi_ref[...])
        bv_ref[...] = jnp.where(c, val, bv_ref[...])

    @pl.when(v == _VT - 1)
    def _finish():
        bv = bv_ref[...]
        bi = bi_ref[...]
        m = jnp.min(bv, axis=1, keepdims=True)
        lif = jnp.min(jnp.where(bv == m, bi, jnp.inf), axis=1, keepdims=True)
        idx_ref[...] = lif.astype(jnp.int32)
        loss_ref[...] = jnp.sum(m).reshape(1, 1, 1)


def _run_argmin(z2, zn, cn2, cb):
    colf = jnp.arange(_VOCAB, dtype=jnp.float32).reshape(1, _VOCAB)
    return pl.pallas_call(
        _argmin_body,
        grid=(_RT, _VT),
        in_specs=[
            pl.BlockSpec((1, _EMBED, _TM), lambda r, v: (r // 2, 0, r % 2)),
            pl.BlockSpec((_TM, 1), lambda r, v: (r, 0)),
            pl.BlockSpec((1, _VOCAB), lambda r, v: (0, 0)),
            pl.BlockSpec((1, _VOCAB), lambda r, v: (0, 0)),
            pl.BlockSpec((_TV, _EMBED), lambda r, v: (v, 0)),
        ],
        out_specs=[
            pl.BlockSpec((_TM, 1), lambda r, v: (r, 0)),
            pl.BlockSpec((1, 1, 1), lambda r, v: (r, 0, 0)),
        ],
        out_shape=[
            jax.ShapeDtypeStruct((_ROWS, 1), jnp.int32),
            jax.ShapeDtypeStruct((_RT, 1, 1), jnp.float32),
        ],
        scratch_shapes=[
            pltpu.VMEM((_TM, 128), jnp.float32),
            pltpu.VMEM((_TM, 128), jnp.float32),
        ],
    )(z2, zn, cn2, colf, cb)


def _ppl_body(idx_ref, lp_ref, out_ref, loss_ref):
    idx = idx_ref[...]
    c = jnp.zeros(idx.shape, jnp.int32)
    for b in range(8):
        c = c + (idx == idx[b:b + 1, :]).astype(jnp.int32)
    p = c.astype(jnp.float32) * 0.125
    ent = jnp.sum(jnp.log(p + 1e-10)) * 0.125
    out_ref[...] = jnp.full((1, 1), jnp.exp(-ent), jnp.float32)
    loss_ref[...] = jnp.full((1, 1), jnp.sum(lp_ref[...]) / float(_ROWS * _EMBED),
                             jnp.float32)


def _run_ppl(idx8, loss_parts):
    return pl.pallas_call(
        _ppl_body,
        out_shape=[
            jax.ShapeDtypeStruct((1, 1), jnp.float32),
            jax.ShapeDtypeStruct((1, 1), jnp.float32),
        ],
    )(idx8, loss_parts)


_NC = 2
_NS = 16
_NW = _NC * _NS
_BPW = _ROWS // _NW
_CH = 128
_NCH = _BPW // _CH

@functools.cache
def _make_sc_gather():
    mesh = plsc.VectorSubcoreMesh(core_axis_name="c", subcore_axis_name="s")

    @functools.partial(
        pl.kernel,
        mesh=mesh,
        out_type=jax.ShapeDtypeStruct((_ROWS, _EMBED), jnp.float32),
        scratch_types=[
            pltpu.VMEM((_CH,), jnp.int32),
            pltpu.VMEM((_CH,), jnp.int32),
            pltpu.VMEM((_CH, _EMBED), jnp.float32),
            pltpu.VMEM((_CH, _EMBED), jnp.float32),
            pltpu.SemaphoreType.DMA,
            pltpu.SemaphoreType.DMA,
        ],
    )
    def _sc_gather(table_hbm, idx_hbm, out_hbm, idx0, idx1, rows0, rows1, sem0, sem1):
        wid = lax.axis_index("s") * _NC + lax.axis_index("c")
        base = wid * _BPW
        idx_bufs = (idx0, idx1)
        row_bufs = (rows0, rows1)
        sems = (sem0, sem1)
        copies = []
        for k in range(_NCH):
            pltpu.sync_copy(idx_hbm.at[pl.ds(base + k * _CH, _CH)], idx_bufs[k])
            copies.append(pltpu.async_copy(table_hbm.at[idx_bufs[k]], row_bufs[k], sems[k]))
        for k in range(_NCH):
            copies[k].wait()
            pltpu.sync_copy(row_bufs[k], out_hbm.at[pl.ds(base + k * _CH, _CH)])

    return _sc_gather


def kernel(z, codebook):
    B, C, H, W = z.shape
    z_flat = jnp.transpose(z, (0, 2, 3, 1)).reshape(B, H * W, C).astype(jnp.float32)
    cb = codebook.astype(jnp.float32)
    zn = jnp.sum(z_flat ** 2, axis=-1, keepdims=True)
    cn = jnp.sum(cb ** 2, axis=-1)

    idx2, loss_parts = _run_argmin(z.reshape(B, C, H * W),
                                   zn.reshape(B * H * W, 1),
                                   cn.reshape(1, _VOCAB), cb)

    idx_flat = idx2.reshape(B * H * W)
    zq_flat = _make_sc_gather()(cb, idx_flat)

    ppl_out, loss_out = _run_ppl(idx2.reshape(B, H * W), loss_parts)
    perplexity = ppl_out[0, 0]
    z_q = jnp.transpose(zq_flat.reshape(B, H, W, C), (0, 3, 1, 2))
    commitment_loss = loss_out[0, 0]
    indices = idx_flat.reshape(B, H, W)
    return z_q, indices, commitment_loss, perplexity
